# baseline (device time: 435096 ns/iter reference)
import jax
import jax.numpy as jnp
from jax import lax
from jax.experimental import pallas as pl
from jax.experimental.pallas import tpu as pltpu

N_DEV = 8
SQ = 2048
SKV = 2048
D_MODEL = 1024
HQ_LOCAL = 8
DH = 128
SCALE = 0.08838834764831843

CHUNK = SQ // N_DEV


def _ring_allreduce(partial):

    def body(p_ref, out_ref, comm_ref, send_sem, rs_recv_sems, ag_recv_sems):
        me = lax.axis_index("i")
        left = (me + N_DEV - 1) % N_DEV
        right = (me + 1) % N_DEV

        barrier_sem = pltpu.get_barrier_semaphore()
        for nbr in [left, right]:
            pl.semaphore_signal(
                barrier_sem, inc=1,
                device_id=(nbr,), device_id_type=pl.DeviceIdType.MESH,
            )
        pl.semaphore_wait(barrier_sem, 2)

        out_ref[...] = p_ref[...]

        for s in range(N_DEV - 1):
            send_idx = (me + 2 * N_DEV - s) % N_DEV
            recv_idx = (me + 2 * N_DEV - s - 1) % N_DEV
            rdma = pltpu.make_async_remote_copy(
                src_ref=out_ref.at[pl.ds(send_idx * CHUNK, CHUNK), :],
                dst_ref=comm_ref.at[s],
                send_sem=send_sem,
                recv_sem=rs_recv_sems.at[s],
                device_id=(right,),
                device_id_type=pl.DeviceIdType.MESH,
            )
            rdma.start()
            rdma.wait()
            out_ref[pl.ds(recv_idx * CHUNK, CHUNK), :] += comm_ref[s]

        for s in range(N_DEV - 1):
            send_idx = (me + 1 + 2 * N_DEV - s) % N_DEV
            rdma = pltpu.make_async_remote_copy(
                src_ref=out_ref.at[pl.ds(send_idx * CHUNK, CHUNK), :],
                dst_ref=out_ref.at[pl.ds(send_idx * CHUNK, CHUNK), :],
                send_sem=send_sem,
                recv_sem=ag_recv_sems.at[s],
                device_id=(right,),
                device_id_type=pl.DeviceIdType.MESH,
            )
            rdma.start()
            rdma.wait()

    return pl.pallas_call(
        body,
        out_shape=jax.ShapeDtypeStruct((SQ, D_MODEL), jnp.float32),
        in_specs=[pl.BlockSpec(memory_space=pltpu.VMEM)],
        out_specs=pl.BlockSpec(memory_space=pltpu.VMEM),
        scratch_shapes=[
            pltpu.VMEM((N_DEV - 1, CHUNK, D_MODEL), jnp.float32),
            pltpu.SemaphoreType.DMA,
            pltpu.SemaphoreType.DMA((N_DEV - 1,)),
            pltpu.SemaphoreType.DMA((N_DEV - 1,)),
        ],
        compiler_params=pltpu.CompilerParams(collective_id=0),
    )(partial)


def kernel(x, Wq, K_ext, V_ext, Wo):
    idx = lax.axis_index("i")

    xb = x[0].astype(jnp.bfloat16)
    Q = jnp.dot(xb, Wq.astype(jnp.bfloat16),
                preferred_element_type=jnp.float32)
    Q = Q.reshape(SQ, HQ_LOCAL, DH)

    K = lax.dynamic_slice_in_dim(K_ext[0], idx * HQ_LOCAL, HQ_LOCAL, axis=1)
    V = lax.dynamic_slice_in_dim(V_ext[0], idx * HQ_LOCAL, HQ_LOCAL, axis=1)

    scores = jnp.einsum(
        "ihd,jhd->hij",
        Q.astype(jnp.bfloat16), K.astype(jnp.bfloat16),
        preferred_element_type=jnp.float32,
    ) * SCALE

    qb = (jnp.arange(SQ) // 64)[:, None]
    kb = (jnp.arange(SKV) // 64)[None, :]
    mask = (qb == kb) | (kb == 0) | ((qb + kb) % 3 == 0)
    scores = jnp.where(mask[None], scores, -1e9)
    w = jax.nn.softmax(scores, axis=-1)

    ctx = jnp.einsum(
        "hij,jhd->ihd",
        w.astype(jnp.bfloat16), V.astype(jnp.bfloat16),
        preferred_element_type=jnp.float32,
    ).reshape(SQ, HQ_LOCAL * DH)

    partial = jnp.dot(ctx.astype(jnp.bfloat16), Wo.astype(jnp.bfloat16),
                      preferred_element_type=jnp.float32)

    out = _ring_allreduce(partial)
    return out[None]


# device time: 408288 ns/iter; 1.0657x vs baseline; 1.0657x over previous
import jax
import jax.numpy as jnp
from jax import lax
from jax.experimental import pallas as pl
from jax.experimental.pallas import tpu as pltpu

N_DEV = 8
SQ = 2048
SKV = 2048
D_MODEL = 1024
HQ_LOCAL = 8
DH = 128
SCALE = 0.08838834764831843

CHUNK = SQ // N_DEV


def _ring_allreduce(partial):

    def body(p_ref, out_ref, comm_ref, send_sem, rs_recv_sems, ag_recv_sems):
        me = lax.axis_index("i")
        left = (me + N_DEV - 1) % N_DEV
        right = (me + 1) % N_DEV

        barrier_sem = pltpu.get_barrier_semaphore()
        for nbr in [left, right]:
            pl.semaphore_signal(
                barrier_sem, inc=1,
                device_id=(nbr,), device_id_type=pl.DeviceIdType.MESH,
            )
        pl.semaphore_wait(barrier_sem, 2)

        out_ref[...] = p_ref[...]

        for s in range(N_DEV - 1):
            send_idx = (me + 2 * N_DEV - s) % N_DEV
            recv_idx = (me + 2 * N_DEV - s - 1) % N_DEV
            rdma = pltpu.make_async_remote_copy(
                src_ref=out_ref.at[pl.ds(send_idx * CHUNK, CHUNK), :],
                dst_ref=comm_ref.at[s],
                send_sem=send_sem,
                recv_sem=rs_recv_sems.at[s],
                device_id=(right,),
                device_id_type=pl.DeviceIdType.MESH,
            )
            rdma.start()
            rdma.wait()
            out_ref[pl.ds(recv_idx * CHUNK, CHUNK), :] += comm_ref[s]

        for s in range(N_DEV - 1):
            send_idx = (me + 1 + 2 * N_DEV - s) % N_DEV
            rdma = pltpu.make_async_remote_copy(
                src_ref=out_ref.at[pl.ds(send_idx * CHUNK, CHUNK), :],
                dst_ref=out_ref.at[pl.ds(send_idx * CHUNK, CHUNK), :],
                send_sem=send_sem,
                recv_sem=ag_recv_sems.at[s],
                device_id=(right,),
                device_id_type=pl.DeviceIdType.MESH,
            )
            rdma.start()
            rdma.wait()

    return pl.pallas_call(
        body,
        out_shape=jax.ShapeDtypeStruct((SQ, D_MODEL), jnp.float32),
        in_specs=[pl.BlockSpec(memory_space=pltpu.VMEM)],
        out_specs=pl.BlockSpec(memory_space=pltpu.VMEM),
        scratch_shapes=[
            pltpu.VMEM((N_DEV - 1, CHUNK, D_MODEL), jnp.float32),
            pltpu.SemaphoreType.DMA,
            pltpu.SemaphoreType.DMA((N_DEV - 1,)),
            pltpu.SemaphoreType.DMA((N_DEV - 1,)),
        ],
        compiler_params=pltpu.CompilerParams(collective_id=0),
    )(partial)


QT = 512
N_QT = SQ // QT


def _attn_body(x_ref, wq_ref, k_ref, v_ref, wo_ref, out_ref):
    qt = pl.program_id(0)
    h = pl.program_id(1)

    xb = x_ref[...].astype(jnp.bfloat16)
    wq = wq_ref[...].astype(jnp.bfloat16)
    q = jnp.dot(xb, wq, preferred_element_type=jnp.float32)

    k = k_ref[0].astype(jnp.bfloat16)
    s = lax.dot_general(
        q.astype(jnp.bfloat16), k,
        dimension_numbers=(((1,), (1,)), ((), ())),
        preferred_element_type=jnp.float32,
    ) * SCALE

    rows = lax.broadcasted_iota(jnp.int32, (QT, SKV), 0) + qt * QT
    cols = lax.broadcasted_iota(jnp.int32, (QT, SKV), 1)
    qb = rows // 64
    kb = cols // 64
    mask = (qb == kb) | (kb == 0) | ((qb + kb) % 3 == 0)
    s = jnp.where(mask, s, -1e9)

    m = jnp.max(s, axis=-1, keepdims=True)
    w = jnp.exp(s - m)
    wsum = jnp.sum(w, axis=-1, keepdims=True)

    v = v_ref[0].astype(jnp.bfloat16)
    ctx = jnp.dot(w.astype(jnp.bfloat16), v,
                  preferred_element_type=jnp.float32) / wsum

    wo = wo_ref[...].astype(jnp.bfloat16)
    contrib = jnp.dot(ctx.astype(jnp.bfloat16), wo,
                      preferred_element_type=jnp.float32)

    @pl.when(h == 0)
    def _():
        out_ref[...] = contrib

    @pl.when(h != 0)
    def _():
        out_ref[...] += contrib


def _attn_partial(x2d, Wq, K, V, Wo):
    grid = (N_QT, HQ_LOCAL)
    return pl.pallas_call(
        _attn_body,
        grid=grid,
        in_specs=[
            pl.BlockSpec((QT, D_MODEL), lambda qt, h: (qt, 0)),
            pl.BlockSpec((D_MODEL, DH), lambda qt, h: (0, h)),
            pl.BlockSpec((1, SKV, DH), lambda qt, h: (h, 0, 0)),
            pl.BlockSpec((1, SKV, DH), lambda qt, h: (h, 0, 0)),
            pl.BlockSpec((DH, D_MODEL), lambda qt, h: (h, 0)),
        ],
        out_specs=pl.BlockSpec((QT, D_MODEL), lambda qt, h: (qt, 0)),
        out_shape=jax.ShapeDtypeStruct((SQ, D_MODEL), jnp.float32),
        compiler_params=pltpu.CompilerParams(
            dimension_semantics=("arbitrary", "arbitrary"),
        ),
    )(x2d, Wq, K, V, Wo)


def kernel(x, Wq, K_ext, V_ext, Wo):
    idx = lax.axis_index("i")

    K = lax.dynamic_slice_in_dim(K_ext[0], idx * HQ_LOCAL, HQ_LOCAL, axis=1)
    V = lax.dynamic_slice_in_dim(V_ext[0], idx * HQ_LOCAL, HQ_LOCAL, axis=1)
    K = jnp.transpose(K, (1, 0, 2))
    V = jnp.transpose(V, (1, 0, 2))

    partial = _attn_partial(x[0], Wq, K, V, Wo)
    out = _ring_allreduce(partial)
    return out[None]
